# speedup vs baseline: 6.1600x; 6.1600x over previous
import functools

import jax
import jax.numpy as jnp
from jax import lax
from jax.experimental import pallas as pl
from jax.experimental.pallas import tpu as pltpu
from jax.experimental.pallas import tpu_sc as plsc

D = 128
SEQ = 200
BATCH = 1024
NC = 2
NS = 16
NW = NC * NS
ROWS_PER_W = BATCH // NW
SPLIT = 128
LANES = 16


def _body(idx_hbm, table_hbm, pe_hbm, out_hbm,
          idx_v, pe_v, buf0, buf1, sem0, sem1):
    cid = lax.axis_index("c")
    sid = lax.axis_index("s")
    wid = sid * NC + cid

    pltpu.sync_copy(idx_hbm.at[wid], idx_v)
    pltpu.sync_copy(pe_hbm.at[pl.ds(0, SEQ)], pe_v)

    bufs = (buf0, buf1)
    sems = (sem0, sem1)

    def start_gather(g):
        b = g % 2
        c0 = pltpu.async_copy(
            table_hbm.at[idx_v.at[g, pl.ds(0, SPLIT)]],
            bufs[b].at[pl.ds(0, SPLIT)], sems[b])
        c1 = pltpu.async_copy(
            table_hbm.at[idx_v.at[g, pl.ds(SPLIT, SEQ - SPLIT)]],
            bufs[b].at[pl.ds(SPLIT, SEQ - SPLIT)], sems[b])
        return c0, c1

    pending = {0: start_gather(0)}

    for g in range(ROWS_PER_W):
        b = g % 2
        buf = bufs[b]
        if g + 1 < ROWS_PER_W:
            pending[g + 1] = start_gather(g + 1)
        for c in pending.pop(g):
            c.wait()

        def add_pe(r, carry):
            for rr in range(2):
                row = r * 2 + rr
                for cc in range(D // LANES):
                    sl = pl.ds(cc * LANES, LANES)
                    plsc.addupdate(buf.at[row, sl], pe_v[row, sl])
            return carry

        lax.fori_loop(0, SEQ // 2, add_pe, 0)

        pltpu.sync_copy(
            buf, out_hbm.at[pl.ds((wid * ROWS_PER_W + g) * SEQ, SEQ)])


_emb = functools.partial(
    pl.kernel,
    out_type=jax.ShapeDtypeStruct((BATCH * SEQ, D), jnp.float32),
    mesh=plsc.VectorSubcoreMesh(core_axis_name="c", subcore_axis_name="s"),
    scratch_types=[
        pltpu.VMEM((ROWS_PER_W, SEQ), jnp.int32),
        pltpu.VMEM((SEQ, D), jnp.float32),
        pltpu.VMEM((SEQ, D), jnp.float32),
        pltpu.VMEM((SEQ, D), jnp.float32),
        pltpu.SemaphoreType.DMA,
        pltpu.SemaphoreType.DMA,
    ],
)(_body)


@jax.jit
def kernel(inputs, table, pos_encoding):
    idx3 = inputs.astype(jnp.int32).reshape(NW, ROWS_PER_W, SEQ)
    out = _emb(idx3, table, pos_encoding)
    return out.reshape(BATCH, SEQ, D)


# speedup vs baseline: 7.1347x; 1.1582x over previous
import functools

import jax
import jax.numpy as jnp
from jax import lax
from jax.experimental import pallas as pl
from jax.experimental.pallas import tpu as pltpu
from jax.experimental.pallas import tpu_sc as plsc

D = 128
SEQ = 200
BATCH = 1024
NC = 2
NS = 16
NW = NC * NS
ROWS_PER_W = BATCH // NW
SPLIT = 128
LANES = 16


NBUF = 3


def _body(idx_hbm, table_hbm, pe_hbm, out_hbm,
          idx_v, pe_v, buf0, buf1, buf2,
          gsem0, gsem1, gsem2, wsem0, wsem1, wsem2):
    cid = lax.axis_index("c")
    sid = lax.axis_index("s")
    wid = sid * NC + cid

    pltpu.sync_copy(idx_hbm.at[wid], idx_v)
    pltpu.sync_copy(pe_hbm.at[pl.ds(0, SEQ)], pe_v)

    bufs = (buf0, buf1, buf2)
    gsems = (gsem0, gsem1, gsem2)
    wsems = (wsem0, wsem1, wsem2)

    def start_gather(g):
        b = g % NBUF
        c0 = pltpu.async_copy(
            table_hbm.at[idx_v.at[g, pl.ds(0, SPLIT)]],
            bufs[b].at[pl.ds(0, SPLIT)], gsems[b])
        c1 = pltpu.async_copy(
            table_hbm.at[idx_v.at[g, pl.ds(SPLIT, SEQ - SPLIT)]],
            bufs[b].at[pl.ds(SPLIT, SEQ - SPLIT)], gsems[b])
        return c0, c1

    gathers = {0: start_gather(0), 1: start_gather(1)}
    writes = {}

    for g in range(ROWS_PER_W):
        b = g % NBUF
        buf = bufs[b]
        for c in gathers.pop(g):
            c.wait()

        def add_pe(r, carry):
            for rr in range(2):
                row = r * 2 + rr
                for cc in range(D // LANES):
                    sl = pl.ds(cc * LANES, LANES)
                    plsc.addupdate(buf.at[row, sl], pe_v[row, sl])
            return carry

        lax.fori_loop(0, SEQ // 2, add_pe, 0)

        writes[g] = pltpu.async_copy(
            buf, out_hbm.at[pl.ds((wid * ROWS_PER_W + g) * SEQ, SEQ)],
            wsems[b])

        if g + 2 < ROWS_PER_W:
            if g - 1 >= 0:
                writes.pop(g - 1).wait()
            gathers[g + 2] = start_gather(g + 2)

    for w in sorted(writes):
        writes.pop(w).wait()


_emb = functools.partial(
    pl.kernel,
    out_type=jax.ShapeDtypeStruct((BATCH * SEQ, D), jnp.float32),
    mesh=plsc.VectorSubcoreMesh(core_axis_name="c", subcore_axis_name="s"),
    scratch_types=[
        pltpu.VMEM((ROWS_PER_W, SEQ), jnp.int32),
        pltpu.VMEM((SEQ, D), jnp.float32),
        pltpu.VMEM((SEQ, D), jnp.float32),
        pltpu.VMEM((SEQ, D), jnp.float32),
        pltpu.VMEM((SEQ, D), jnp.float32),
        pltpu.SemaphoreType.DMA,
        pltpu.SemaphoreType.DMA,
        pltpu.SemaphoreType.DMA,
        pltpu.SemaphoreType.DMA,
        pltpu.SemaphoreType.DMA,
        pltpu.SemaphoreType.DMA,
    ],
)(_body)


@jax.jit
def kernel(inputs, table, pos_encoding):
    idx3 = inputs.astype(jnp.int32).reshape(NW, ROWS_PER_W, SEQ)
    out = _emb(idx3, table, pos_encoding)
    return out.reshape(BATCH, SEQ, D)
